# baseline (speedup 1.0000x reference)
import functools

import jax
import jax.numpy as jnp
from jax import lax
from jax.experimental import pallas as pl
from jax.experimental.pallas import tpu as pltpu
from jax.experimental.pallas import tpu_sc as plsc

N = 10000
E = 320000
G = 16

_PC = (3.5075520536942406e-06, 0.999792435728606, -0.49697791116761014,
       0.31459053537083104, -0.18878267362071732, 0.08172680837495,
       -0.017208061121084715)

_NSUB = 16
_K = 80
_EPT = E // _NSUB
_NPAD = 10240
_NPT = _NPAD // _NSUB
_ZR = 128
_TW = 128
_HH = _TW // 2


_sc_mesh = plsc.VectorSubcoreMesh(core_axis_name="c", subcore_axis_name="s")


@functools.partial(
    pl.kernel,
    out_type=jax.ShapeDtypeStruct((2, _NPAD, _TW), jnp.float32),
    mesh=_sc_mesh,
    scratch_types=[
        pltpu.VMEM((_K,), jnp.int32),
        pltpu.VMEM((_K, _TW), jnp.float32),
        pltpu.VMEM((_K, _TW), jnp.float32),
        pltpu.VMEM((_K, _TW), jnp.float32),
        pltpu.VMEM((_ZR, _TW), jnp.float32),
        pltpu.VMEM_SHARED((_NPAD, _TW), jnp.float32),
    ],
)
def _sc_layer(u_all, ae_all, dst_i,
              out, idxd, U, Eb, M, zbuf, agg):
    c = lax.axis_index("c")
    s = lax.axis_index("s")

    def _zrow(i, carry):
        for j in range(_TW // 16):
            zbuf[i, pl.ds(j * 16, 16)] = jnp.zeros((16,), jnp.float32)
        return carry
    lax.fori_loop(0, _ZR, _zrow, 0)

    def _mrow(k, carry):
        for j in range(_HH // 16):
            M[k, pl.ds(_HH + j * 16, 16)] = jnp.zeros((16,), jnp.float32)
        return carry
    lax.fori_loop(0, _K, _mrow, 0)
    for b in range(_NPT // _ZR):
        pltpu.sync_copy(zbuf, agg.at[pl.ds(s * _NPT + b * _ZR, _ZR)])
    plsc.subcore_barrier()

    base_e = s * _EPT

    def _chunk(i, carry):
        e0 = base_e + i * _K
        off = c * E + e0
        pltpu.sync_copy(dst_i.at[pl.ds(e0, _K)], idxd)
        pltpu.sync_copy(u_all.at[pl.ds(off, _K)], U)
        pltpu.sync_copy(ae_all.at[pl.ds(off, _K)], Eb)

        def _edge(k, inner):
            for j in range(_HH // 16):
                lo = pl.ds(j * 16, 16)
                hi = pl.ds(_HH + j * 16, 16)
                gp = U[k, lo] + Eb[k, lo]
                cp = U[k, hi] + Eb[k, hi]
                g = 1.0 / (1.0 + jnp.exp(-gp))
                t = jnp.exp(-jnp.abs(cp))
                p = jnp.full((16,), _PC[6], jnp.float32)
                for coef in (_PC[5], _PC[4], _PC[3], _PC[2], _PC[1], _PC[0]):
                    p = p * t + coef
                sp = jnp.maximum(cp, 0.0) + p
                M[k, lo] = g * sp
            return inner
        lax.fori_loop(0, _K, _edge, 0)

        pltpu.sync_copy(M, agg.at[idxd], add=True)
        return carry
    lax.fori_loop(0, _EPT // _K, _chunk, 0)

    plsc.subcore_barrier()
    pltpu.sync_copy(agg.at[pl.ds(s * _NPT, _NPT)],
                    out.at[c, pl.ds(s * _NPT, _NPT)])


_BN = 1000
_BE = 2000


def _node_tables_body(h_ref, wd_ref, ws_ref, od, os_):
    hb = h_ref[...]
    od[...] = jnp.dot(hb, wd_ref[0], preferred_element_type=jnp.float32)
    os_[...] = jnp.dot(hb, ws_ref[0], preferred_element_type=jnp.float32)


def _node_tables(h, wd, ws):
    C = h.shape[1]
    return pl.pallas_call(
        _node_tables_body,
        grid=(2, N // _BN),
        in_specs=[pl.BlockSpec((_BN, C), lambda c, i: (i, 0)),
                  pl.BlockSpec((1, C, _TW), lambda c, i: (c, 0, 0)),
                  pl.BlockSpec((1, C, _TW), lambda c, i: (c, 0, 0))],
        out_specs=[pl.BlockSpec((_BN, _TW),
                                lambda c, i: (c * (N // _BN) + i, 0))] * 2,
        out_shape=[jax.ShapeDtypeStruct((2 * N, _TW), jnp.float32)] * 2,
    )(h, wd, ws)


def _node_tables_res_body(h_ref, q0_ref, q1_ref, q2_ref, q3_ref,
                          wd_ref, ws_ref, od, os_, oh):
    hb = h_ref[...] + jnp.concatenate(
        [q0_ref[...], q1_ref[...], q2_ref[...], q3_ref[...]], axis=1)
    hb = jnp.maximum(hb, 0.0)
    oh[...] = hb
    od[...] = jnp.dot(hb, wd_ref[0], preferred_element_type=jnp.float32)
    os_[...] = jnp.dot(hb, ws_ref[0], preferred_element_type=jnp.float32)


def _node_tables_res(h, quarters, wd, ws):
    C = h.shape[1]
    return pl.pallas_call(
        _node_tables_res_body,
        grid=(2, N // _BN),
        in_specs=[pl.BlockSpec((_BN, C), lambda c, i: (i, 0))]
        + [pl.BlockSpec((_BN, _HH), lambda c, i: (i, 0))] * 4
        + [pl.BlockSpec((1, C, _TW), lambda c, i: (c, 0, 0)),
           pl.BlockSpec((1, C, _TW), lambda c, i: (c, 0, 0))],
        out_specs=[pl.BlockSpec((_BN, _TW),
                                lambda c, i: (c * (N // _BN) + i, 0))] * 2
        + [pl.BlockSpec((_BN, C), lambda c, i: (i, 0))],
        out_shape=[jax.ShapeDtypeStruct((2 * N, _TW), jnp.float32)] * 2
        + [jax.ShapeDtypeStruct((N, C), jnp.float32)],
    )(h, *quarters, wd, ws)


def _lin_body(x_ref, q0_ref, q1_ref, w_ref, b_ref, o_ref):
    hb = x_ref[...] + jnp.concatenate([q0_ref[...], q1_ref[...]], axis=1)
    hb = jnp.maximum(hb, 0.0)
    o_ref[...] = (jnp.dot(hb, w_ref[...], preferred_element_type=jnp.float32)
                  + b_ref[...])


def _lin(x, q0, q1, w, b):
    C = x.shape[1]
    H = w.shape[1]
    return pl.pallas_call(
        _lin_body,
        grid=(N // _BN,),
        in_specs=[pl.BlockSpec((_BN, C), lambda i: (i, 0)),
                  pl.BlockSpec((_BN, _HH), lambda i: (i, 0)),
                  pl.BlockSpec((_BN, _HH), lambda i: (i, 0)),
                  pl.BlockSpec((C, H), lambda i: (0, 0)),
                  pl.BlockSpec((1, H), lambda i: (0, 0))],
        out_specs=pl.BlockSpec((_BN, H), lambda i: (i, 0)),
        out_shape=jax.ShapeDtypeStruct((N, H), jnp.float32),
    )(x, q0, q1, w, b)


def _edge_tables_body(ea_ref, w_ref, b_ref, o_ref):
    o_ref[...] = (jnp.dot(ea_ref[...], w_ref[0],
                          preferred_element_type=jnp.float32)
                  + b_ref[0])


def _edge_tables(ea, w, b):
    DE = ea.shape[1]
    return pl.pallas_call(
        _edge_tables_body,
        grid=(2, E // _BE),
        in_specs=[pl.BlockSpec((_BE, DE), lambda c, i: (i, 0)),
                  pl.BlockSpec((1, DE, _TW), lambda c, i: (c, 0, 0)),
                  pl.BlockSpec((1, 1, _TW), lambda c, i: (c, 0, 0))],
        out_specs=pl.BlockSpec((_BE, _TW),
                               lambda c, i: (c * (E // _BE) + i, 0)),
        out_shape=jax.ShapeDtypeStruct((2 * E, _TW), jnp.float32),
    )(ea, w, b)


def _pool_body(h_ref, q0_ref, q1_ref, q2_ref, q3_ref, b_ref, sums_ref):
    i = pl.program_id(0)

    @pl.when(i == 0)
    def _():
        sums_ref[...] = jnp.zeros_like(sums_ref)

    h3 = h_ref[...] + jnp.concatenate(
        [q0_ref[...], q1_ref[...], q2_ref[...], q3_ref[...]], axis=1)
    h3 = jnp.maximum(h3, 0.0)
    bb = b_ref[0]
    onehot = (bb == lax.broadcasted_iota(jnp.int32, (G, _BN), 0))
    onehot = onehot.astype(jnp.float32)
    sums_ref[...] += jnp.dot(onehot, h3, preferred_element_type=jnp.float32)


def _pool(h, quarters, batch_blocks):
    C = h.shape[1]
    return pl.pallas_call(
        _pool_body,
        grid=(N // _BN,),
        in_specs=[pl.BlockSpec((_BN, C), lambda i: (i, 0))]
        + [pl.BlockSpec((_BN, _HH), lambda i: (i, 0))] * 4
        + [pl.BlockSpec((1, 1, _BN), lambda i: (i, 0, 0))],
        out_specs=pl.BlockSpec((G, C), lambda i: (0, 0)),
        out_shape=jax.ShapeDtypeStruct((G, C), jnp.float32),
    )(h, *quarters, batch_blocks)


def _probe_body(sums_ref, batch_ref, wp_ref, bp_ref, out_ref):
    bb = batch_ref[...]
    onehot = (bb == lax.broadcasted_iota(jnp.int32, (G, N), 0))
    counts = jnp.sum(onehot.astype(jnp.float32), axis=1, keepdims=True)
    pooled = sums_ref[...] / jnp.maximum(counts, 1.0)
    out_ref[...] = (jnp.sum(pooled * wp_ref[...], axis=1, keepdims=True)
                    + bp_ref[0])


def _probe(sums, batch_row, wp, bp):
    return pl.pallas_call(
        _probe_body,
        out_shape=jax.ShapeDtypeStruct((G, 1), jnp.float32),
    )(sums, batch_row, wp, bp)


def _prep_layer(Wf, bf, Ws, bs, D):
    Wf_d, Wf_s, Wf_e = Wf[:, :D], Wf[:, D:2 * D], Wf[:, 2 * D:]
    Ws_d, Ws_s, Ws_e = Ws[:, :D], Ws[:, D:2 * D], Ws[:, 2 * D:]
    preps = []
    for i in range(D // _TW):
        wd, wsrc, wedg, bedg = [], [], [], []
        for h in (0, 1):
            sl = slice(i * _TW + h * _HH, i * _TW + (h + 1) * _HH)
            wd.append(jnp.concatenate([Wf_d[sl], Ws_d[sl]], axis=0).T)
            wsrc.append(jnp.concatenate([Wf_s[sl], Ws_s[sl]], axis=0).T)
            wedg.append(jnp.concatenate([Wf_e[sl], Ws_e[sl]], axis=0).T)
            bedg.append(jnp.concatenate([bf[sl], bs[sl]])[None, :])
        preps.append((jnp.stack(wd),
                      jnp.stack(wsrc),
                      jnp.stack(wedg),
                      jnp.stack(bedg)))
    return preps


def _run_sc_call(h, prep, ea, dst2, src2, dst):
    wd, ws, we, be = prep
    ad_all, as_all = _node_tables(h, wd, ws)
    ae_all = _edge_tables(ea, we, be)
    u_all = ad_all[dst2] + as_all[src2]
    agg = _sc_layer(u_all, ae_all, dst)
    return agg[0, :N, :_HH], agg[1, :N, :_HH]


def kernel(x, edge_index, edge_attr, batch, Wf1, bf1, Ws1, bs1, Wlin, blin,
           Wf2, bf2, Ws2, bs2, Wf3, bf3, Ws3, bs3, Wp, bp):
    src = edge_index[0]
    dst = edge_index[1]
    dst2 = jnp.concatenate([dst, dst + N])
    src2 = jnp.concatenate([src, src + N])
    batch_blocks = batch.reshape(N // _BN, 1, _BN)
    batch_row = batch.reshape(1, N)

    p1 = _prep_layer(Wf1, bf1, Ws1, bs1, 128)
    p2 = _prep_layer(Wf2, bf2, Ws2, bs2, 256)
    p3 = _prep_layer(Wf3, bf3, Ws3, bs3, 256)

    q0, q1 = _run_sc_call(x, p1[0], edge_attr, dst2, src2, dst)

    hlin = _lin(x, q0, q1, Wlin.T, blin[None, :])

    q2a = _run_sc_call(hlin, p2[0], edge_attr, dst2, src2, dst)
    q2b = _run_sc_call(hlin, p2[1], edge_attr, dst2, src2, dst)
    quarters2 = [q2a[0], q2a[1], q2b[0], q2b[1]]

    ad_a, as_a, h2 = _node_tables_res(hlin, quarters2, p3[0][0], p3[0][1])
    ae_a = _edge_tables(edge_attr, p3[0][2], p3[0][3])
    agg3a = _sc_layer(ad_a[dst2] + as_a[src2], ae_a, dst)

    q3b = _run_sc_call(h2, p3[1], edge_attr, dst2, src2, dst)

    quarters3 = [agg3a[0, :N, :_HH], agg3a[1, :N, :_HH], q3b[0], q3b[1]]

    sums = _pool(h2, quarters3, batch_blocks)
    return _probe(sums, batch_row, Wp, bp)


# speedup vs baseline: 2.0746x; 2.0746x over previous
import functools

import jax
import jax.numpy as jnp
from jax import lax
from jax.experimental import pallas as pl
from jax.experimental.pallas import tpu as pltpu
from jax.experimental.pallas import tpu_sc as plsc

N = 10000
E = 320000
G = 16

_PC = (3.5075520536942406e-06, 0.999792435728606, -0.49697791116761014,
       0.31459053537083104, -0.18878267362071732, 0.08172680837495,
       -0.017208061121084715)

_NSUB = 16
_K = 80
_EPT = E // _NSUB
_NPAD = 10240
_NPT = _NPAD // _NSUB
_ZR = 128
_TW = 128
_HH = _TW // 2


_sc_mesh = plsc.VectorSubcoreMesh(core_axis_name="c", subcore_axis_name="s")


def _make_sc_layer(call_idx, wfull):
    @functools.partial(
        pl.kernel,
        out_type=jax.ShapeDtypeStruct((2, _NPAD, _TW), jnp.float32),
        mesh=_sc_mesh,
        scratch_types=[
            pltpu.VMEM((_K,), jnp.int32),
            pltpu.VMEM((_K, _TW), jnp.float32),
            pltpu.VMEM((_K, _TW), jnp.float32),
            pltpu.VMEM((_K, _TW), jnp.float32),
            pltpu.VMEM((_ZR, _TW), jnp.float32),
            pltpu.VMEM_SHARED((_NPAD, _TW), jnp.float32),
        ],
    )
    def sc_layer(u_all, ae_all, dst_i,
                 out, idxd, U, Eb, M, zbuf, agg):
        c = lax.axis_index("c")
        s = lax.axis_index("s")

        def _zrow(i, carry):
            for j in range(_TW // 16):
                zbuf[i, pl.ds(j * 16, 16)] = jnp.zeros((16,), jnp.float32)
            return carry
        lax.fori_loop(0, _ZR, _zrow, 0)

        def _mrow(k, carry):
            for j in range(_HH // 16):
                M[k, pl.ds(_HH + j * 16, 16)] = jnp.zeros((16,), jnp.float32)
            return carry
        lax.fori_loop(0, _K, _mrow, 0)
        for b in range(_NPT // _ZR):
            pltpu.sync_copy(zbuf, agg.at[pl.ds(s * _NPT + b * _ZR, _ZR)])
        plsc.subcore_barrier()

        base_e = s * _EPT

        def _chunk(i, carry):
            e0 = base_e + i * _K
            col = (2 * call_idx + c) * _TW
            pltpu.sync_copy(dst_i.at[pl.ds(e0, _K)], idxd)
            pltpu.sync_copy(u_all.at[pl.ds(e0, _K), pl.ds(col, _TW)], U)
            pltpu.sync_copy(ae_all.at[pl.ds(e0, _K), pl.ds(col, _TW)], Eb)

            def _edge(k, inner):
                for j in range(_HH // 16):
                    lo = pl.ds(j * 16, 16)
                    hi = pl.ds(_HH + j * 16, 16)
                    gp = U[k, lo] + Eb[k, lo]
                    cp = U[k, hi] + Eb[k, hi]
                    g = 1.0 / (1.0 + jnp.exp(-gp))
                    t = jnp.exp(-jnp.abs(cp))
                    p = jnp.full((16,), _PC[6], jnp.float32)
                    for coef in (_PC[5], _PC[4], _PC[3], _PC[2], _PC[1], _PC[0]):
                        p = p * t + coef
                    sp = jnp.maximum(cp, 0.0) + p
                    M[k, lo] = g * sp
                return inner
            lax.fori_loop(0, _K, _edge, 0)

            pltpu.sync_copy(M, agg.at[idxd], add=True)
            return carry
        lax.fori_loop(0, _EPT // _K, _chunk, 0)

        plsc.subcore_barrier()
        pltpu.sync_copy(agg.at[pl.ds(s * _NPT, _NPT)],
                        out.at[c, pl.ds(s * _NPT, _NPT)])

    return sc_layer


_sc_call_0 = _make_sc_layer(0, 256)
_sc_call_0w = _make_sc_layer(0, 512)
_sc_call_1w = _make_sc_layer(1, 512)


_BN = 1000
_BE = 2000


def _node_tables_body(h_ref, wd_ref, ws_ref, od, os_):
    hb = h_ref[...]
    od[...] = jnp.dot(hb, wd_ref[...], preferred_element_type=jnp.float32)
    os_[...] = jnp.dot(hb, ws_ref[...], preferred_element_type=jnp.float32)


def _node_tables(h, wd, ws):
    C = h.shape[1]
    W = wd.shape[1]
    return pl.pallas_call(
        _node_tables_body,
        grid=(N // _BN,),
        in_specs=[pl.BlockSpec((_BN, C), lambda i: (i, 0)),
                  pl.BlockSpec((C, W), lambda i: (0, 0)),
                  pl.BlockSpec((C, W), lambda i: (0, 0))],
        out_specs=[pl.BlockSpec((_BN, W), lambda i: (i, 0))] * 2,
        out_shape=[jax.ShapeDtypeStruct((N, W), jnp.float32)] * 2,
    )(h, wd, ws)


def _node_tables_res_body(h_ref, q0_ref, q1_ref, q2_ref, q3_ref,
                          wd_ref, ws_ref, od, os_, oh):
    hb = h_ref[...] + jnp.concatenate(
        [q0_ref[...], q1_ref[...], q2_ref[...], q3_ref[...]], axis=1)
    hb = jnp.maximum(hb, 0.0)
    oh[...] = hb
    od[...] = jnp.dot(hb, wd_ref[...], preferred_element_type=jnp.float32)
    os_[...] = jnp.dot(hb, ws_ref[...], preferred_element_type=jnp.float32)


def _node_tables_res(h, quarters, wd, ws):
    C = h.shape[1]
    W = wd.shape[1]
    return pl.pallas_call(
        _node_tables_res_body,
        grid=(N // _BN,),
        in_specs=[pl.BlockSpec((_BN, C), lambda i: (i, 0))]
        + [pl.BlockSpec((_BN, _HH), lambda i: (i, 0))] * 4
        + [pl.BlockSpec((C, W), lambda i: (0, 0)),
           pl.BlockSpec((C, W), lambda i: (0, 0))],
        out_specs=[pl.BlockSpec((_BN, W), lambda i: (i, 0))] * 2
        + [pl.BlockSpec((_BN, C), lambda i: (i, 0))],
        out_shape=[jax.ShapeDtypeStruct((N, W), jnp.float32)] * 2
        + [jax.ShapeDtypeStruct((N, C), jnp.float32)],
    )(h, *quarters, wd, ws)


def _lin_body(x_ref, q0_ref, q1_ref, w_ref, b_ref, o_ref):
    hb = x_ref[...] + jnp.concatenate([q0_ref[...], q1_ref[...]], axis=1)
    hb = jnp.maximum(hb, 0.0)
    o_ref[...] = (jnp.dot(hb, w_ref[...], preferred_element_type=jnp.float32)
                  + b_ref[...])


def _lin(x, q0, q1, w, b):
    C = x.shape[1]
    H = w.shape[1]
    return pl.pallas_call(
        _lin_body,
        grid=(N // _BN,),
        in_specs=[pl.BlockSpec((_BN, C), lambda i: (i, 0)),
                  pl.BlockSpec((_BN, _HH), lambda i: (i, 0)),
                  pl.BlockSpec((_BN, _HH), lambda i: (i, 0)),
                  pl.BlockSpec((C, H), lambda i: (0, 0)),
                  pl.BlockSpec((1, H), lambda i: (0, 0))],
        out_specs=pl.BlockSpec((_BN, H), lambda i: (i, 0)),
        out_shape=jax.ShapeDtypeStruct((N, H), jnp.float32),
    )(x, q0, q1, w, b)


def _edge_tables_body(ea_ref, w_ref, b_ref, o_ref):
    o_ref[...] = (jnp.dot(ea_ref[...], w_ref[...],
                          preferred_element_type=jnp.float32)
                  + b_ref[...])


def _edge_tables(ea, w, b):
    DE = ea.shape[1]
    W = w.shape[1]
    return pl.pallas_call(
        _edge_tables_body,
        grid=(E // _BE,),
        in_specs=[pl.BlockSpec((_BE, DE), lambda i: (i, 0)),
                  pl.BlockSpec((DE, W), lambda i: (0, 0)),
                  pl.BlockSpec((1, W), lambda i: (0, 0))],
        out_specs=pl.BlockSpec((_BE, W), lambda i: (i, 0)),
        out_shape=jax.ShapeDtypeStruct((E, W), jnp.float32),
    )(ea, w, b)


def _pool_body(h_ref, q0_ref, q1_ref, q2_ref, q3_ref, b_ref, sums_ref):
    i = pl.program_id(0)

    @pl.when(i == 0)
    def _():
        sums_ref[...] = jnp.zeros_like(sums_ref)

    h3 = h_ref[...] + jnp.concatenate(
        [q0_ref[...], q1_ref[...], q2_ref[...], q3_ref[...]], axis=1)
    h3 = jnp.maximum(h3, 0.0)
    bb = b_ref[0]
    onehot = (bb == lax.broadcasted_iota(jnp.int32, (G, _BN), 0))
    onehot = onehot.astype(jnp.float32)
    sums_ref[...] += jnp.dot(onehot, h3, preferred_element_type=jnp.float32)


def _pool(h, quarters, batch_blocks):
    C = h.shape[1]
    return pl.pallas_call(
        _pool_body,
        grid=(N // _BN,),
        in_specs=[pl.BlockSpec((_BN, C), lambda i: (i, 0))]
        + [pl.BlockSpec((_BN, _HH), lambda i: (i, 0))] * 4
        + [pl.BlockSpec((1, 1, _BN), lambda i: (i, 0, 0))],
        out_specs=pl.BlockSpec((G, C), lambda i: (0, 0)),
        out_shape=jax.ShapeDtypeStruct((G, C), jnp.float32),
    )(h, *quarters, batch_blocks)


def _probe_body(sums_ref, batch_ref, wp_ref, bp_ref, out_ref):
    bb = batch_ref[...]
    onehot = (bb == lax.broadcasted_iota(jnp.int32, (G, N), 0))
    counts = jnp.sum(onehot.astype(jnp.float32), axis=1, keepdims=True)
    pooled = sums_ref[...] / jnp.maximum(counts, 1.0)
    out_ref[...] = (jnp.sum(pooled * wp_ref[...], axis=1, keepdims=True)
                    + bp_ref[0])


def _probe(sums, batch_row, wp, bp):
    return pl.pallas_call(
        _probe_body,
        out_shape=jax.ShapeDtypeStruct((G, 1), jnp.float32),
    )(sums, batch_row, wp, bp)


def _prep_layer(Wf, bf, Ws, bs, D):
    Wf_d, Wf_s, Wf_e = Wf[:, :D], Wf[:, D:2 * D], Wf[:, 2 * D:]
    Ws_d, Ws_s, Ws_e = Ws[:, :D], Ws[:, D:2 * D], Ws[:, 2 * D:]
    wd, wsrc, wedg, bedg = [], [], [], []
    for i in range(D // _TW):
        for h in (0, 1):
            sl = slice(i * _TW + h * _HH, i * _TW + (h + 1) * _HH)
            wd.append(jnp.concatenate([Wf_d[sl], Ws_d[sl]], axis=0).T)
            wsrc.append(jnp.concatenate([Wf_s[sl], Ws_s[sl]], axis=0).T)
            wedg.append(jnp.concatenate([Wf_e[sl], Ws_e[sl]], axis=0).T)
            bedg.append(jnp.concatenate([bf[sl], bs[sl]]))
    return (jnp.concatenate(wd, axis=1),
            jnp.concatenate(wsrc, axis=1),
            jnp.concatenate(wedg, axis=1),
            jnp.concatenate(bedg)[None, :])


def _run_sc_call(h, prep, ea, dst, src, sc_calls):
    wd, ws, we, be = prep
    ad_full, as_full = _node_tables(h, wd, ws)
    ae_full = _edge_tables(ea, we, be)
    u_full = ad_full[dst] + as_full[src]
    quarters = []
    for sc in sc_calls:
        agg = sc(u_full, ae_full, dst)
        quarters.append(agg[0, :N, :_HH])
        quarters.append(agg[1, :N, :_HH])
    return quarters


def kernel(x, edge_index, edge_attr, batch, Wf1, bf1, Ws1, bs1, Wlin, blin,
           Wf2, bf2, Ws2, bs2, Wf3, bf3, Ws3, bs3, Wp, bp):
    src = edge_index[0]
    dst = edge_index[1]
    batch_blocks = batch.reshape(N // _BN, 1, _BN)
    batch_row = batch.reshape(1, N)

    p1 = _prep_layer(Wf1, bf1, Ws1, bs1, 128)
    p2 = _prep_layer(Wf2, bf2, Ws2, bs2, 256)
    p3 = _prep_layer(Wf3, bf3, Ws3, bs3, 256)

    q0, q1 = _run_sc_call(x, p1, edge_attr, dst, src, [_sc_call_0])

    hlin = _lin(x, q0, q1, Wlin.T, blin[None, :])

    quarters2 = _run_sc_call(hlin, p2, edge_attr, dst, src,
                             [_sc_call_0w, _sc_call_1w])

    ad_full, as_full, h2 = _node_tables_res(hlin, quarters2, p3[0], p3[1])
    ae_full = _edge_tables(edge_attr, p3[2], p3[3])
    u_full = ad_full[dst] + as_full[src]
    quarters3 = []
    for sc in (_sc_call_0w, _sc_call_1w):
        agg = sc(u_full, ae_full, dst)
        quarters3.append(agg[0, :N, :_HH])
        quarters3.append(agg[1, :N, :_HH])

    sums = _pool(h2, quarters3, batch_blocks)
    return _probe(sums, batch_row, Wp, bp)
